# baseline (speedup 1.0000x reference)
import functools

import jax
import jax.numpy as jnp
from jax import lax
from jax.experimental import pallas as pl
from jax.experimental.pallas import tpu as pltpu
from jax.experimental.pallas import tpu_sc as plsc

_VOCAB = 1000000
_EMBED = 64
_BATCH = 16384

_MM_BLK = 2048


@functools.lru_cache(maxsize=None)
def _build_gather():
    info = plsc.get_sparse_core_info()
    nw = info.num_cores * info.num_subcores
    bpw = _BATCH // nw
    nl = info.num_lanes

    mesh = plsc.VectorSubcoreMesh(core_axis_name="c", subcore_axis_name="s")

    @functools.partial(
        pl.kernel,
        mesh=mesh,
        out_type=jax.ShapeDtypeStruct((_BATCH * _EMBED,), jnp.float32),
        scratch_types=[
            pltpu.VMEM((bpw,), jnp.int32),
            pltpu.VMEM((bpw * _EMBED,), jnp.int32),
            pltpu.VMEM((bpw * _EMBED,), jnp.float32),
            pltpu.SemaphoreType.DMA,
        ],
        compiler_params=pltpu.CompilerParams(use_tc_tiling_on_sc=False),
    )
    def gather_sc(tflat_hbm, idx_hbm, g_hbm, idx_v, eidx_v, rows_v, sem):
        wid = lax.axis_index("s") * info.num_cores + lax.axis_index("c")
        base = wid * bpw
        pltpu.sync_copy(idx_hbm.at[pl.ds(base, bpw)], idx_v)

        lanes = lax.iota(jnp.int32, nl)

        def body(g, carry):
            vec = idx_v[pl.ds(g * nl, nl)]
            for j in range(nl):
                t = g * nl + j
                col = vec[j]
                for r in range(_EMBED // nl):
                    eidx_v[pl.ds(t * _EMBED + r * nl, nl)] = (
                        (lanes + (r * nl)) * _VOCAB + col
                    )
            return carry

        lax.fori_loop(0, bpw // nl, body, 0)
        pltpu.async_copy(tflat_hbm.at[eidx_v], rows_v, sem).wait()
        pltpu.sync_copy(rows_v, g_hbm.at[pl.ds(base * _EMBED, bpw * _EMBED)])

    return gather_sc


def _linear_body(x_ref, w_ref, b_ref, o_ref):
    o_ref[...] = lax.dot_general(
        x_ref[...],
        w_ref[...],
        dimension_numbers=(((1,), (1,)), ((), ())),
        preferred_element_type=jnp.float32,
    ) + b_ref[...]


@jax.jit
def kernel(input_ids, table, W, b):
    tflat = table.T.reshape(-1)
    g = _build_gather()(tflat, input_ids).reshape(_BATCH, _EMBED)
    out = pl.pallas_call(
        _linear_body,
        grid=(_BATCH // _MM_BLK,),
        in_specs=[
            pl.BlockSpec((_MM_BLK, _EMBED), lambda i: (i, 0)),
            pl.BlockSpec((_EMBED, _EMBED), lambda i: (0, 0)),
            pl.BlockSpec((1, _EMBED), lambda i: (0, 0)),
        ],
        out_specs=pl.BlockSpec((_MM_BLK, _EMBED), lambda i: (i, 0)),
        out_shape=jax.ShapeDtypeStruct((_BATCH, _EMBED), jnp.float32),
    )(g, W, b[None, :])
    return out
